# Initial kernel scaffold; baseline (speedup 1.0000x reference)
import jax
import jax.numpy as jnp
from jax.experimental import pallas as pl


def kernel(input_bits, connections, memory):
    raise NotImplementedError("write your pallas kernel here")


# speedup vs baseline: 1.0801x; 1.0801x over previous
import functools

import jax
import jax.numpy as jnp
from jax import lax
from jax.experimental import pallas as pl
from jax.experimental.pallas import tpu as pltpu
from jax.experimental.pallas import tpu_sc as plsc

_TOTAL_BITS = 4096
_NEURONS = 8192
_NBITS = 12
_BATCH = 1024
_LANES = 16
_WORDS = _BATCH // 4
_GROUPS = _WORDS // _LANES


@functools.lru_cache(maxsize=None)
def _build_sc_kernel():
    info = plsc.get_sparse_core_info()
    nc, ns = info.num_cores, info.num_subcores
    nw = nc * ns
    npt = _NEURONS // nw
    mesh = plsc.VectorSubcoreMesh(core_axis_name="c", subcore_axis_name="s")

    @functools.partial(
        pl.kernel,
        mesh=mesh,
        compiler_params=pltpu.CompilerParams(needs_layout_passes=False),
        out_type=jax.ShapeDtypeStruct((_NEURONS, _BATCH), jnp.float32),
        scratch_types=[
            pltpu.VMEM((npt, 16), jnp.int32),
            pltpu.VMEM((16, _WORDS), jnp.int32),
            pltpu.VMEM((2 ** _NBITS,), jnp.float32),
            pltpu.VMEM((_BATCH,), jnp.float32),
            pltpu.SemaphoreType.DMA,
        ],
    )
    def ram_kernel(bits_hbm, conn_hbm, mem_hbm, out_hbm,
                   conn_v, rows_v, mrow_v, orow_v, sem):
        wid = lax.axis_index("s") * nc + lax.axis_index("c")
        n0 = wid * npt
        pltpu.sync_copy(conn_hbm.at[pl.ds(n0, npt)], conn_v)

        def body(i, carry):
            pltpu.async_copy(bits_hbm.at[conn_v.at[i]], rows_v, sem).wait()
            pltpu.sync_copy(mem_hbm.at[n0 + i], mrow_v)
            for t in range(_GROUPS):
                lo = jnp.zeros((_LANES,), jnp.int32)
                hi = jnp.zeros((_LANES,), jnp.int32)
                for k in range(_NBITS):
                    w = rows_v[k, pl.ds(t * _LANES, _LANES)]
                    if k < 6:
                        lo = lo + (w << k)
                    else:
                        hi = hi + (w << (k - 6))
                for j in range(4):
                    addr = ((lo >> (8 * j)) & 0x3F) | (((hi >> (8 * j)) & 0x3F) << 6)
                    vals = plsc.load_gather(mrow_v, [addr])
                    orow_v[pl.ds(t * 64 + j * _LANES, _LANES)] = vals
            pltpu.sync_copy(orow_v, out_hbm.at[n0 + i])
            return carry

        lax.fori_loop(0, npt, body, 0)

    return ram_kernel


def kernel(input_bits, connections, memory):
    bits8_t = input_bits.astype(jnp.int8).T
    bits_packed = jax.lax.bitcast_convert_type(
        bits8_t.reshape(_TOTAL_BITS, _WORDS, 4), jnp.int32)
    conn_p = jnp.pad(connections, ((0, 0), (0, 16 - _NBITS)))
    vals = _build_sc_kernel()(bits_packed, conn_p, memory)
    vals = vals.reshape(_NEURONS, _GROUPS, 4, _LANES).swapaxes(2, 3)
    return vals.reshape(_NEURONS, _BATCH).T > 0.5


# speedup vs baseline: 5.0720x; 4.6958x over previous
import functools

import jax
import jax.numpy as jnp
from jax import lax
from jax.experimental import pallas as pl
from jax.experimental.pallas import tpu as pltpu
from jax.experimental.pallas import tpu_sc as plsc

_TOTAL_BITS = 4096
_NEURONS = 8192
_NBITS = 12
_BATCH = 1024
_LANES = 16
_WORDS = _BATCH // 4
_GROUPS = _WORDS // _LANES
_CHUNK = 4


@functools.lru_cache(maxsize=None)
def _build_sc_kernel():
    info = plsc.get_sparse_core_info()
    nc, ns = info.num_cores, info.num_subcores
    nw = nc * ns
    npt = _NEURONS // nw
    nchunks = npt // _CHUNK
    crow = _CHUNK * _NBITS
    mesh = plsc.VectorSubcoreMesh(core_axis_name="c", subcore_axis_name="s")

    @functools.partial(
        pl.kernel,
        mesh=mesh,
        compiler_params=pltpu.CompilerParams(needs_layout_passes=False),
        out_type=jax.ShapeDtypeStruct((_NEURONS, _BATCH), jnp.float32),
        scratch_types=[
            pltpu.VMEM((npt * _NBITS,), jnp.int32),
            pltpu.VMEM((crow, _WORDS), jnp.int32),
            pltpu.VMEM((crow, _WORDS), jnp.int32),
            pltpu.VMEM((_CHUNK, 2 ** _NBITS), jnp.float32),
            pltpu.VMEM((_CHUNK, 2 ** _NBITS), jnp.float32),
            pltpu.VMEM((_CHUNK, _BATCH), jnp.float32),
            pltpu.VMEM((_CHUNK, _BATCH), jnp.float32),
            pltpu.SemaphoreType.DMA,
            pltpu.SemaphoreType.DMA,
            pltpu.SemaphoreType.DMA,
            pltpu.SemaphoreType.DMA,
            pltpu.SemaphoreType.DMA,
            pltpu.SemaphoreType.DMA,
        ],
    )
    def ram_kernel(bits_hbm, conn_hbm, mem_hbm, out_hbm,
                   conn_v, rows_a, rows_b, mem_a, mem_b, out_a, out_b,
                   sin_a, sin_b, smem_a, smem_b, sout_a, sout_b):
        wid = lax.axis_index("s") * nc + lax.axis_index("c")
        n0 = wid * npt
        pltpu.sync_copy(conn_hbm.at[pl.ds(n0 * _NBITS, npt * _NBITS)], conn_v)

        rows = (rows_a, rows_b)
        mem = (mem_a, mem_b)
        out = (out_a, out_b)
        sin = (sin_a, sin_b)
        smem = (smem_a, smem_b)
        sout = (sout_a, sout_b)

        def issue_in(ci, b):
            pltpu.make_async_copy(
                bits_hbm.at[conn_v.at[pl.ds(ci * crow, crow)]], rows[b], sin[b]
            ).start()
            pltpu.make_async_copy(
                mem_hbm.at[pl.ds(n0 + ci * _CHUNK, _CHUNK)], mem[b], smem[b]
            ).start()

        def wait_in(b):
            pltpu.make_async_copy(
                bits_hbm.at[pl.ds(0, crow)], rows[b], sin[b]).wait()
            pltpu.make_async_copy(
                mem_hbm.at[pl.ds(0, _CHUNK)], mem[b], smem[b]).wait()

        def issue_out(ci, b):
            pltpu.make_async_copy(
                out[b], out_hbm.at[pl.ds(n0 + ci * _CHUNK, _CHUNK)], sout[b]
            ).start()

        def wait_out(b):
            pltpu.make_async_copy(
                out[b], out_hbm.at[pl.ds(0, _CHUNK)], sout[b]).wait()

        def compute(b):
            for q in range(_CHUNK):
                qvec = jnp.full((_LANES,), q, jnp.int32)
                for t in range(_GROUPS):
                    lo = jnp.zeros((_LANES,), jnp.int32)
                    hi = jnp.zeros((_LANES,), jnp.int32)
                    for k in range(_NBITS):
                        w = rows[b][q * _NBITS + k, pl.ds(t * _LANES, _LANES)]
                        if k < 6:
                            lo = lo + (w << k)
                        else:
                            hi = hi + (w << (k - 6))
                    for j in range(4):
                        addr = ((lo >> (8 * j)) & 0x3F) | (((hi >> (8 * j)) & 0x3F) << 6)
                        vals = plsc.load_gather(mem[b], [qvec, addr])
                        out[b][q, pl.ds(t * 64 + j * _LANES, _LANES)] = vals

        issue_in(0, 0)

        def body(h, carry):
            c0 = h * 2
            issue_in(c0 + 1, 1)
            wait_in(0)

            @pl.when(h > 0)
            def _():
                wait_out(0)

            compute(0)
            issue_out(c0, 0)

            @pl.when(h < nchunks // 2 - 1)
            def _():
                issue_in(c0 + 2, 0)

            wait_in(1)

            @pl.when(h > 0)
            def _():
                wait_out(1)

            compute(1)
            issue_out(c0 + 1, 1)
            return carry

        lax.fori_loop(0, nchunks // 2, body, 0)
        wait_out(0)
        wait_out(1)

    return ram_kernel


def kernel(input_bits, connections, memory):
    bits8_t = input_bits.astype(jnp.int8).T
    bits_packed = jax.lax.bitcast_convert_type(
        bits8_t.reshape(_TOTAL_BITS, _WORDS, 4), jnp.int32)
    conn_flat = connections.reshape(-1)
    vals = _build_sc_kernel()(bits_packed, conn_flat, memory)
    vals = vals.reshape(_NEURONS, _GROUPS, 4, _LANES).swapaxes(2, 3)
    return vals.reshape(_NEURONS, _BATCH).T > 0.5
